# baseline (speedup 1.0000x reference)
import functools

import jax
import jax.numpy as jnp
from jax import lax
from jax.experimental import pallas as pl
from jax.experimental.pallas import tpu as pltpu
from jax.experimental.pallas import tpu_sc as plsc

BATCH = 16384
D = 64
NC = 2
NS = 16
NW = NC * NS
BPW = BATCH // NW
L = 16
CH = 128
NCH = BPW // CH
NBUF = 2


def _cf_body(uid_hbm, mid_hbm, ut_hbm, mt_hbm, wb_hbm, out_hbm,
             uidx, midx, utile, mtile, uring, mring, wv, outv, usem, msem):
    wid = lax.axis_index("s") * NC + lax.axis_index("c")
    base = wid * BPW

    pltpu.sync_copy(uid_hbm.at[pl.ds(base, BPW)], uidx)
    pltpu.sync_copy(mid_hbm.at[pl.ds(base, BPW)], midx)
    pltpu.sync_copy(wb_hbm, wv)

    for j in range(BPW // L):
        sl = pl.ds(j * L, L)
        utile[sl] = uidx[sl] >> 1
        mtile[sl] = midx[sl] >> 1

    lane = lax.iota(jnp.int32, L)
    last_lane = lane == (L - 1)

    def issue_chunk(c, par):
        pltpu.async_copy(
            ut_hbm.at[utile.at[pl.ds(c * CH, CH)]], uring.at[par], usem)
        pltpu.async_copy(
            mt_hbm.at[mtile.at[pl.ds(c * CH, CH)]], mring.at[par], msem)

    def wait_chunk(par):
        pltpu.make_async_copy(
            ut_hbm.at[utile.at[pl.ds(0, CH)]], uring.at[par], usem).wait()
        pltpu.make_async_copy(
            mt_hbm.at[mtile.at[pl.ds(0, CH)]], mring.at[par], msem).wait()

    issue_chunk(0, 0)

    w = [wv[pl.ds(k * L, L)] for k in range(2 * D // L)]
    bvec = wv[pl.ds(2 * D, L)]

    def chunk(g, carry):
        par = lax.rem(g, NBUF)
        wait_chunk(par)

        @pl.when(g + 1 < NCH)
        def _():
            issue_chunk(g + 1, lax.rem(g + 1, NBUF))

        row0 = g * CH
        for b in range(CH // L):
            uoff = (uidx[pl.ds(row0 + b * L, L)] & 1) * D
            moff = (midx[pl.ds(row0 + b * L, L)] & 1) * D
            for r in range(L):
                uo = uoff[r]
                mo = moff[r]
                idx = b * L + r
                acc = None
                for k in range(D // L):
                    uv = uring[par, idx, pl.ds(uo + k * L, L)] * w[k]
                    mv = mring[par, idx, pl.ds(mo + k * L, L)] * w[D // L + k]
                    t = uv + mv
                    acc = t if acc is None else acc + t
                s = plsc.cumsum(acc) + bvec
                plsc.store_scatter(
                    outv, [jnp.full((L,), row0 + idx, jnp.int32)], s,
                    mask=last_lane,
                )
        return carry

    lax.fori_loop(0, NCH, chunk, 0)

    pltpu.sync_copy(outv, out_hbm.at[pl.ds(base, BPW)])


@jax.jit
def _cf_call(user_ids, movie_ids, ut2, mt2, wb):
    mesh = plsc.VectorSubcoreMesh(core_axis_name="c", subcore_axis_name="s")
    f = functools.partial(
        pl.kernel,
        mesh=mesh,
        compiler_params=pltpu.CompilerParams(needs_layout_passes=False),
        out_type=jax.ShapeDtypeStruct((BATCH,), jnp.float32),
        scratch_types=[
            pltpu.VMEM((BPW,), jnp.int32),
            pltpu.VMEM((BPW,), jnp.int32),
            pltpu.VMEM((BPW,), jnp.int32),
            pltpu.VMEM((BPW,), jnp.int32),
            pltpu.VMEM((NBUF, CH, 2 * D), jnp.float32),
            pltpu.VMEM((NBUF, CH, 2 * D), jnp.float32),
            pltpu.VMEM((2 * D + L,), jnp.float32),
            pltpu.VMEM((BPW,), jnp.float32),
            pltpu.SemaphoreType.DMA,
            pltpu.SemaphoreType.DMA,
        ],
    )(_cf_body)
    return f(user_ids, movie_ids, ut2, mt2, wb)


def kernel(user_ids, movie_ids, user_table, movie_table, W, b):
    wb = jnp.concatenate(
        [W.reshape(2 * D), jnp.broadcast_to(b.reshape(1), (L,))]
    )
    return _cf_call(
        user_ids.astype(jnp.int32), movie_ids.astype(jnp.int32),
        user_table.reshape(-1, 2 * D), movie_table.reshape(-1, 2 * D), wb,
    )
